# baseline (speedup 1.0000x reference)
import functools

import jax
import jax.numpy as jnp
from jax import lax
from jax.experimental import pallas as pl
from jax.experimental.pallas import tpu as pltpu
from jax.experimental.pallas import tpu_sc as plsc

N = 10000
E = 160000
F_IN = 256
HID = 128
HID2 = 64
NUM_CLASSES = 2
EPS = 1e-5

NC = 2
NS = 16
NW = NC * NS

E_PAD = 163840
N_PAD = 10240
D1 = HID
D2 = HID2

_MESH = dict(core_axis_name="c", subcore_axis_name="s", num_cores=NC,
             num_subcores=NS)


def _make_sc_agg(d, chb, nbuf):
    nch = E_PAD // NW // chb
    rows_per_tile = N_PAD // NS
    reps = rows_per_tile // chb

    @functools.partial(
        pl.kernel,
        out_type=jax.ShapeDtypeStruct((NC, N_PAD, d), jnp.float32),
        mesh=plsc.VectorSubcoreMesh(**_MESH),
        compiler_params=pltpu.CompilerParams(use_tc_tiling_on_sc=False),
        scratch_types=[
            pltpu.VMEM((nch, chb), jnp.int32),
            pltpu.VMEM((nch, chb), jnp.int32),
            pltpu.VMEM((nbuf, chb, d), jnp.float32),
            pltpu.VMEM_SHARED((N_PAD, d), jnp.float32),
            [pltpu.SemaphoreType.DMA] * nbuf,
        ],
    )
    def k(table, srcp, dstp, out, sidx, didx, rows, acc, sems):
        c = lax.axis_index("c")
        s = lax.axis_index("s")
        wid = c * NS + s

        zv = jnp.zeros((16,), jnp.float32)

        def zero_row(r, _):
            for kk in range(d // 16):
                rows[0, r, pl.ds(kk * 16, 16)] = zv
            return 0

        lax.fori_loop(0, chb, zero_row, 0)
        tile_base = s * rows_per_tile
        for q in range(reps):
            pltpu.sync_copy(rows.at[0], acc.at[pl.ds(tile_base + q * chb, chb)])
        plsc.subcore_barrier()

        pltpu.sync_copy(srcp.at[wid], sidx)
        pltpu.sync_copy(dstp.at[wid], didx)

        def gather(j):
            return pltpu.async_copy(table.at[sidx.at[j]], rows.at[j % nbuf],
                                    sems[j % nbuf])

        def scatter(j):
            return pltpu.async_copy(rows.at[j % nbuf], acc.at[didx.at[j]],
                                    sems[j % nbuf], add=True)

        gd = [None] * nch
        sd = [None] * nch
        for b in range(nbuf - 1):
            gd[b] = gather(b)
        for j in range(nch):
            if j >= 1:
                sd[j - 1].wait()
            if j + nbuf - 1 < nch:
                gd[j + nbuf - 1] = gather(j + nbuf - 1)
            gd[j].wait()
            sd[j] = scatter(j)
        sd[nch - 1].wait()
        plsc.subcore_barrier()

        pltpu.sync_copy(acc.at[pl.ds(tile_base, rows_per_tile)],
                        out.at[c, pl.ds(tile_base, rows_per_tile)])

    return k


_sc_agg_d1 = _make_sc_agg(D1, 64, 4)
_sc_agg_d2 = _make_sc_agg(D2, 128, 4)


def _make_sc_cnt():
    chb = 128
    nch = E_PAD // NW // chb
    rows_per_tile = N_PAD // NS
    reps = rows_per_tile // chb

    @functools.partial(
        pl.kernel,
        out_type=jax.ShapeDtypeStruct((NC, N_PAD, 16), jnp.float32),
        mesh=plsc.VectorSubcoreMesh(**_MESH),
        compiler_params=pltpu.CompilerParams(use_tc_tiling_on_sc=False),
        scratch_types=[
            pltpu.VMEM((nch, chb), jnp.int32),
            pltpu.VMEM((2, chb, 16), jnp.float32),
            pltpu.VMEM_SHARED((N_PAD, 16), jnp.float32),
            pltpu.SemaphoreType.DMA,
        ],
    )
    def k(dstp, out, didx, bufs, acc, sem):
        c = lax.axis_index("c")
        s = lax.axis_index("s")
        wid = c * NS + s
        zv = jnp.zeros((16,), jnp.float32)
        ov = jnp.ones((16,), jnp.float32)

        def fill(r, _):
            bufs[0, r, pl.ds(0, 16)] = zv
            bufs[1, r, pl.ds(0, 16)] = ov
            return 0

        lax.fori_loop(0, chb, fill, 0)
        tile_base = s * rows_per_tile
        for q in range(reps):
            pltpu.sync_copy(bufs.at[0], acc.at[pl.ds(tile_base + q * chb, chb)])
        plsc.subcore_barrier()
        pltpu.sync_copy(dstp.at[wid], didx)
        descs = [pltpu.async_copy(bufs.at[1], acc.at[didx.at[j]], sem, add=True)
                 for j in range(nch)]
        for dsc in descs:
            dsc.wait()
        plsc.subcore_barrier()
        pltpu.sync_copy(acc.at[pl.ds(tile_base, rows_per_tile)],
                        out.at[c, pl.ds(tile_base, rows_per_tile)])

    return k


_sc_cnt = _make_sc_cnt()

_BN = 2000


def _tc1_body(x_ref, wl_ref, wr_ref, bl_ref, t_ref, r_ref):
    xb = x_ref[...]
    p = lax.dot_general(xb, wl_ref[...], (((1,), (1,)), ((), ())),
                        preferred_element_type=jnp.float32)
    r = lax.dot_general(xb, wr_ref[...], (((1,), (1,)), ((), ())),
                        preferred_element_type=jnp.float32) + bl_ref[...]
    t_ref[...] = p
    r_ref[...] = r


def _tc1(x, w1l, w1r, b1l):
    return pl.pallas_call(
        _tc1_body,
        grid=(N // _BN,),
        in_specs=[
            pl.BlockSpec((_BN, F_IN), lambda i: (i, 0)),
            pl.BlockSpec((HID, F_IN), lambda i: (0, 0)),
            pl.BlockSpec((HID, F_IN), lambda i: (0, 0)),
            pl.BlockSpec((1, HID), lambda i: (0, 0)),
        ],
        out_specs=[
            pl.BlockSpec((_BN, D1), lambda i: (i, 0)),
            pl.BlockSpec((_BN, HID), lambda i: (i, 0)),
        ],
        out_shape=[
            jax.ShapeDtypeStruct((N, D1), jnp.float32),
            jax.ShapeDtypeStruct((N, HID), jnp.float32),
        ],
    )(x, w1l, w1r, b1l.reshape(1, HID))


def _tc2_body(a_ref, cb_ref, r1_ref, g_ref, bt_ref, m_ref, v_ref, wl_ref,
              wr_ref, b2_ref, p2_ref, r2_ref, ic_ref):
    st = a_ref[0] + a_ref[1]
    cnt = cb_ref[0, :, 0:1] + cb_ref[1, :, 0:1]
    invc = 1.0 / jnp.maximum(cnt, 1.0)
    h = st[:, :HID] * invc + r1_ref[...]
    h = (h - m_ref[...]) * lax.rsqrt(v_ref[...] + EPS) * g_ref[...] + bt_ref[...]
    h = jnp.maximum(h, 0.0)
    p2_ref[...] = lax.dot_general(h, wl_ref[...], (((1,), (1,)), ((), ())),
                                  preferred_element_type=jnp.float32)
    r2_ref[...] = lax.dot_general(h, wr_ref[...], (((1,), (1,)), ((), ())),
                                  preferred_element_type=jnp.float32) + b2_ref[...]
    ic_ref[...] = jnp.broadcast_to(invc, (invc.shape[0], 8))


def _tc2(acc1p, cntp, r1, g, bt, m, v, w2l, w2r, b2l):
    return pl.pallas_call(
        _tc2_body,
        grid=(N // _BN,),
        in_specs=[
            pl.BlockSpec((NC, _BN, D1), lambda i: (0, i, 0)),
            pl.BlockSpec((NC, _BN, 16), lambda i: (0, i, 0)),
            pl.BlockSpec((_BN, HID), lambda i: (i, 0)),
            pl.BlockSpec((1, HID), lambda i: (0, 0)),
            pl.BlockSpec((1, HID), lambda i: (0, 0)),
            pl.BlockSpec((1, HID), lambda i: (0, 0)),
            pl.BlockSpec((1, HID), lambda i: (0, 0)),
            pl.BlockSpec((HID2, HID), lambda i: (0, 0)),
            pl.BlockSpec((HID2, HID), lambda i: (0, 0)),
            pl.BlockSpec((1, HID2), lambda i: (0, 0)),
        ],
        out_specs=[
            pl.BlockSpec((_BN, D2), lambda i: (i, 0)),
            pl.BlockSpec((_BN, HID2), lambda i: (i, 0)),
            pl.BlockSpec((_BN, 8), lambda i: (i, 0)),
        ],
        out_shape=[
            jax.ShapeDtypeStruct((N, D2), jnp.float32),
            jax.ShapeDtypeStruct((N, HID2), jnp.float32),
            jax.ShapeDtypeStruct((N, 8), jnp.float32),
        ],
    )(acc1p, cntp, r1, g.reshape(1, HID), bt.reshape(1, HID),
      m.reshape(1, HID), v.reshape(1, HID), w2l, w2r, b2l.reshape(1, HID2))


def _tc3_body(a_ref, r2_ref, ic_ref, wc_ref, bc_ref, o_ref):
    s2 = a_ref[0] + a_ref[1]
    h2 = jnp.maximum(s2 * ic_ref[:, 0:1] + r2_ref[...], 0.0)
    lg = lax.dot_general(h2, wc_ref[...], (((1,), (1,)), ((), ())),
                         preferred_element_type=jnp.float32) + bc_ref[...]
    mx = jnp.max(lg, axis=1, keepdims=True)
    e = lg - mx
    o_ref[...] = e - jnp.log(jnp.sum(jnp.exp(e), axis=1, keepdims=True))


def _tc3(acc2p, r2, ic, wc, bc):
    return pl.pallas_call(
        _tc3_body,
        grid=(N // _BN,),
        in_specs=[
            pl.BlockSpec((NC, _BN, D2), lambda i: (0, i, 0)),
            pl.BlockSpec((_BN, HID2), lambda i: (i, 0)),
            pl.BlockSpec((_BN, 8), lambda i: (i, 0)),
            pl.BlockSpec((NUM_CLASSES, HID2), lambda i: (0, 0)),
            pl.BlockSpec((1, NUM_CLASSES), lambda i: (0, 0)),
        ],
        out_specs=pl.BlockSpec((_BN, NUM_CLASSES), lambda i: (i, 0)),
        out_shape=jax.ShapeDtypeStruct((N, NUM_CLASSES), jnp.float32),
    )(acc2p, r2, ic, wc, bc.reshape(1, NUM_CLASSES))


def kernel(x, edge_index, W1l, b1l, W1r, bn_gamma, bn_beta, bn_mean, bn_var,
           W2l, b2l, W2r, Wc, bc):
    src = edge_index[0]
    dst = edge_index[1]
    pad = E_PAD - E
    pad_ar = jnp.arange(pad, dtype=jnp.int32)
    srcf = jnp.concatenate([src, pad_ar % N])
    dstf = jnp.concatenate([dst, N + pad_ar % (N_PAD - N)])
    srcp1 = srcf.reshape(NW, E_PAD // NW // 64, 64)
    dstp1 = dstf.reshape(NW, E_PAD // NW // 64, 64)
    srcp2 = srcf.reshape(NW, E_PAD // NW // 128, 128)
    dstp2 = dstf.reshape(NW, E_PAD // NW // 128, 128)

    cntp = _sc_cnt(dstp2)
    table1, r1 = _tc1(x, W1l, W1r, b1l)
    acc1p = _sc_agg_d1(table1, srcp1, dstp1)
    p2, r2, ic = _tc2(acc1p, cntp, r1, bn_gamma, bn_beta, bn_mean, bn_var,
                      W2l, W2r, b2l)
    acc2p = _sc_agg_d2(p2, srcp2, dstp2)
    return _tc3(acc2p, r2, ic, Wc, bc)


# speedup vs baseline: 1.0185x; 1.0185x over previous
import functools

import jax
import jax.numpy as jnp
from jax import lax
from jax.experimental import pallas as pl
from jax.experimental.pallas import tpu as pltpu
from jax.experimental.pallas import tpu_sc as plsc

N = 10000
E = 160000
F_IN = 256
HID = 128
HID2 = 64
NUM_CLASSES = 2
EPS = 1e-5

NC = 2
NS = 16
NW = NC * NS

E_PAD = 163840
N_PAD = 10240
D1 = HID
D2 = HID2

_MESH = dict(core_axis_name="c", subcore_axis_name="s", num_cores=NC,
             num_subcores=NS)


def _make_sc_agg(d, chb, nbuf):
    nch = E_PAD // NW // chb
    rows_per_tile = N_PAD // NS
    reps = rows_per_tile // chb

    @functools.partial(
        pl.kernel,
        out_type=jax.ShapeDtypeStruct((NC, N_PAD, d), jnp.float32),
        mesh=plsc.VectorSubcoreMesh(**_MESH),
        compiler_params=pltpu.CompilerParams(use_tc_tiling_on_sc=False),
        scratch_types=[
            pltpu.VMEM((nch, chb), jnp.int32),
            pltpu.VMEM((nch, chb), jnp.int32),
            pltpu.VMEM((nbuf, chb, d), jnp.float32),
            pltpu.VMEM_SHARED((N_PAD, d), jnp.float32),
            [pltpu.SemaphoreType.DMA] * nbuf,
        ],
    )
    def k(table, srcp, dstp, out, sidx, didx, rows, acc, sems):
        c = lax.axis_index("c")
        s = lax.axis_index("s")
        wid = c * NS + s

        zv = jnp.zeros((16,), jnp.float32)

        def zero_row(r, _):
            for kk in range(d // 16):
                rows[0, r, pl.ds(kk * 16, 16)] = zv
            return 0

        lax.fori_loop(0, chb, zero_row, 0)
        tile_base = s * rows_per_tile
        for q in range(reps):
            pltpu.sync_copy(rows.at[0], acc.at[pl.ds(tile_base + q * chb, chb)])
        plsc.subcore_barrier()

        pltpu.sync_copy(srcp.at[wid], sidx)
        pltpu.sync_copy(dstp.at[wid], didx)

        def gather(j):
            return pltpu.async_copy(table.at[sidx.at[j]], rows.at[j % nbuf],
                                    sems[j % nbuf])

        def scatter(j):
            return pltpu.async_copy(rows.at[j % nbuf], acc.at[didx.at[j]],
                                    sems[j % nbuf], add=True)

        gd = [None] * nch
        sd = [None] * nch
        for b in range(nbuf - 1):
            gd[b] = gather(b)
        for j in range(nch):
            if j >= 1:
                sd[j - 1].wait()
            if j + nbuf - 1 < nch:
                gd[j + nbuf - 1] = gather(j + nbuf - 1)
            gd[j].wait()
            sd[j] = scatter(j)
        sd[nch - 1].wait()
        plsc.subcore_barrier()

        pltpu.sync_copy(acc.at[pl.ds(tile_base, rows_per_tile)],
                        out.at[c, pl.ds(tile_base, rows_per_tile)])

    return k


_sc_agg_d1 = _make_sc_agg(D1, 64, 4)
_sc_agg_d2 = _make_sc_agg(D2, 128, 4)


def _make_sc_cnt():
    chb = 128
    nch = E_PAD // NW // chb
    rows_per_tile = N_PAD // NS
    reps = rows_per_tile // chb

    @functools.partial(
        pl.kernel,
        out_type=jax.ShapeDtypeStruct((NC, N_PAD, 16), jnp.float32),
        mesh=plsc.VectorSubcoreMesh(**_MESH),
        compiler_params=pltpu.CompilerParams(use_tc_tiling_on_sc=False),
        scratch_types=[
            pltpu.VMEM((nch, chb), jnp.int32),
            pltpu.VMEM((2, chb, 16), jnp.float32),
            pltpu.VMEM_SHARED((N_PAD, 16), jnp.float32),
            pltpu.SemaphoreType.DMA,
        ],
    )
    def k(dstp, out, didx, bufs, acc, sem):
        c = lax.axis_index("c")
        s = lax.axis_index("s")
        wid = c * NS + s
        zv = jnp.zeros((16,), jnp.float32)
        ov = jnp.ones((16,), jnp.float32)

        def fill(r, _):
            bufs[0, r, pl.ds(0, 16)] = zv
            bufs[1, r, pl.ds(0, 16)] = ov
            return 0

        lax.fori_loop(0, chb, fill, 0)
        tile_base = s * rows_per_tile
        for q in range(reps):
            pltpu.sync_copy(bufs.at[0], acc.at[pl.ds(tile_base + q * chb, chb)])
        plsc.subcore_barrier()
        pltpu.sync_copy(dstp.at[wid], didx)
        descs = [pltpu.async_copy(bufs.at[1], acc.at[didx.at[j]], sem, add=True)
                 for j in range(nch)]
        for dsc in descs:
            dsc.wait()
        plsc.subcore_barrier()
        pltpu.sync_copy(acc.at[pl.ds(tile_base, rows_per_tile)],
                        out.at[c, pl.ds(tile_base, rows_per_tile)])

    return k


_sc_cnt = _make_sc_cnt()

_BN = 5000


def _tc1_body(x_ref, wl_ref, wr_ref, bl_ref, t_ref, r_ref):
    xb = x_ref[...]
    p = lax.dot_general(xb, wl_ref[...], (((1,), (1,)), ((), ())),
                        preferred_element_type=jnp.float32)
    r = lax.dot_general(xb, wr_ref[...], (((1,), (1,)), ((), ())),
                        preferred_element_type=jnp.float32) + bl_ref[...]
    t_ref[...] = p
    r_ref[...] = r


def _tc1(x, w1l, w1r, b1l):
    return pl.pallas_call(
        _tc1_body,
        grid=(N // _BN,),
        in_specs=[
            pl.BlockSpec((_BN, F_IN), lambda i: (i, 0)),
            pl.BlockSpec((HID, F_IN), lambda i: (0, 0)),
            pl.BlockSpec((HID, F_IN), lambda i: (0, 0)),
            pl.BlockSpec((1, HID), lambda i: (0, 0)),
        ],
        out_specs=[
            pl.BlockSpec((_BN, D1), lambda i: (i, 0)),
            pl.BlockSpec((_BN, HID), lambda i: (i, 0)),
        ],
        out_shape=[
            jax.ShapeDtypeStruct((N, D1), jnp.float32),
            jax.ShapeDtypeStruct((N, HID), jnp.float32),
        ],
    )(x, w1l, w1r, b1l.reshape(1, HID))


def _tc2_body(a_ref, cb_ref, r1_ref, g_ref, bt_ref, m_ref, v_ref, wl_ref,
              wr_ref, b2_ref, p2_ref, r2_ref, ic_ref):
    st = a_ref[0] + a_ref[1]
    cnt = cb_ref[0, :, 0:1] + cb_ref[1, :, 0:1]
    invc = 1.0 / jnp.maximum(cnt, 1.0)
    h = st[:, :HID] * invc + r1_ref[...]
    h = (h - m_ref[...]) * lax.rsqrt(v_ref[...] + EPS) * g_ref[...] + bt_ref[...]
    h = jnp.maximum(h, 0.0)
    p2_ref[...] = lax.dot_general(h, wl_ref[...], (((1,), (1,)), ((), ())),
                                  preferred_element_type=jnp.float32)
    r2_ref[...] = lax.dot_general(h, wr_ref[...], (((1,), (1,)), ((), ())),
                                  preferred_element_type=jnp.float32) + b2_ref[...]
    ic_ref[...] = jnp.broadcast_to(invc, (invc.shape[0], 8))


def _tc2(acc1p, cntp, r1, g, bt, m, v, w2l, w2r, b2l):
    return pl.pallas_call(
        _tc2_body,
        grid=(N // _BN,),
        in_specs=[
            pl.BlockSpec((NC, _BN, D1), lambda i: (0, i, 0)),
            pl.BlockSpec((NC, _BN, 16), lambda i: (0, i, 0)),
            pl.BlockSpec((_BN, HID), lambda i: (i, 0)),
            pl.BlockSpec((1, HID), lambda i: (0, 0)),
            pl.BlockSpec((1, HID), lambda i: (0, 0)),
            pl.BlockSpec((1, HID), lambda i: (0, 0)),
            pl.BlockSpec((1, HID), lambda i: (0, 0)),
            pl.BlockSpec((HID2, HID), lambda i: (0, 0)),
            pl.BlockSpec((HID2, HID), lambda i: (0, 0)),
            pl.BlockSpec((1, HID2), lambda i: (0, 0)),
        ],
        out_specs=[
            pl.BlockSpec((_BN, D2), lambda i: (i, 0)),
            pl.BlockSpec((_BN, HID2), lambda i: (i, 0)),
            pl.BlockSpec((_BN, 8), lambda i: (i, 0)),
        ],
        out_shape=[
            jax.ShapeDtypeStruct((N, D2), jnp.float32),
            jax.ShapeDtypeStruct((N, HID2), jnp.float32),
            jax.ShapeDtypeStruct((N, 8), jnp.float32),
        ],
    )(acc1p, cntp, r1, g.reshape(1, HID), bt.reshape(1, HID),
      m.reshape(1, HID), v.reshape(1, HID), w2l, w2r, b2l.reshape(1, HID2))


def _tc3_body(a_ref, r2_ref, ic_ref, wc_ref, bc_ref, o_ref):
    s2 = a_ref[0] + a_ref[1]
    h2 = jnp.maximum(s2 * ic_ref[:, 0:1] + r2_ref[...], 0.0)
    lg = lax.dot_general(h2, wc_ref[...], (((1,), (1,)), ((), ())),
                         preferred_element_type=jnp.float32) + bc_ref[...]
    mx = jnp.max(lg, axis=1, keepdims=True)
    e = lg - mx
    o_ref[...] = e - jnp.log(jnp.sum(jnp.exp(e), axis=1, keepdims=True))


def _tc3(acc2p, r2, ic, wc, bc):
    return pl.pallas_call(
        _tc3_body,
        grid=(N // _BN,),
        in_specs=[
            pl.BlockSpec((NC, _BN, D2), lambda i: (0, i, 0)),
            pl.BlockSpec((_BN, HID2), lambda i: (i, 0)),
            pl.BlockSpec((_BN, 8), lambda i: (i, 0)),
            pl.BlockSpec((NUM_CLASSES, HID2), lambda i: (0, 0)),
            pl.BlockSpec((1, NUM_CLASSES), lambda i: (0, 0)),
        ],
        out_specs=pl.BlockSpec((_BN, NUM_CLASSES), lambda i: (i, 0)),
        out_shape=jax.ShapeDtypeStruct((N, NUM_CLASSES), jnp.float32),
    )(acc2p, r2, ic, wc, bc.reshape(1, NUM_CLASSES))


def kernel(x, edge_index, W1l, b1l, W1r, bn_gamma, bn_beta, bn_mean, bn_var,
           W2l, b2l, W2r, Wc, bc):
    src = edge_index[0]
    dst = edge_index[1]
    pad = E_PAD - E
    pad_ar = jnp.arange(pad, dtype=jnp.int32)
    srcf = jnp.concatenate([src, pad_ar % N])
    dstf = jnp.concatenate([dst, N + pad_ar % (N_PAD - N)])
    srcp1 = srcf.reshape(NW, E_PAD // NW // 64, 64)
    dstp1 = dstf.reshape(NW, E_PAD // NW // 64, 64)
    srcp2 = srcf.reshape(NW, E_PAD // NW // 128, 128)
    dstp2 = dstf.reshape(NW, E_PAD // NW // 128, 128)

    cntp = _sc_cnt(dstp2)
    table1, r1 = _tc1(x, W1l, W1r, b1l)
    acc1p = _sc_agg_d1(table1, srcp1, dstp1)
    p2, r2, ic = _tc2(acc1p, cntp, r1, bn_gamma, bn_beta, bn_mean, bn_var,
                      W2l, W2r, b2l)
    acc2p = _sc_agg_d2(p2, srcp2, dstp2)
    return _tc3(acc2p, r2, ic, Wc, bc)
